# baseline (speedup 1.0000x reference)
import functools

import jax
import jax.numpy as jnp
import numpy as np
from jax import lax
from jax.experimental import pallas as pl
from jax.experimental.pallas import tpu as pltpu
from jax.experimental.pallas import tpu_sc as plsc

F = 128
F3 = 384
NRBF = 20
CUT = 5.0
N = 10000
E = 160000

NC, NS = 2, 16
P = 3
NW = NC * NS
RT = 112
NOUT = P * NW * RT
SLAB_ROWS = 128
DUMMY = RT
CE = 2000
K = 16
CW = F3 + F
WMW = F3 + 3 * F
WM2 = F3 + F


def _phiv_body(s_ref, w1_ref, b1_ref, w2_ref, b2_ref, vx_ref, vy_ref, vz_ref,
               out_ref):
    x = s_ref[...]
    h = jnp.dot(x, w1_ref[...], preferred_element_type=jnp.float32) + b1_ref[...]
    h = h * jax.nn.sigmoid(h)
    out_ref[:, :F3] = (
        jnp.dot(h, w2_ref[...], preferred_element_type=jnp.float32) + b2_ref[...]
    )
    out_ref[:, F3:F3 + F] = vx_ref[...]
    out_ref[:, F3 + F:F3 + 2 * F] = vy_ref[...]
    out_ref[:, F3 + 2 * F:] = vz_ref[...]


def _compute_phiv(s, W1, b1, W2, b2, vtx, vty, vtz):
    blk = 2000
    return pl.pallas_call(
        _phiv_body,
        grid=(N // blk,),
        in_specs=[
            pl.BlockSpec((blk, F), lambda i: (i, 0)),
            pl.BlockSpec((F, F), lambda i: (0, 0)),
            pl.BlockSpec((1, F), lambda i: (0, 0)),
            pl.BlockSpec((F, F3), lambda i: (0, 0)),
            pl.BlockSpec((1, F3), lambda i: (0, 0)),
            pl.BlockSpec((blk, F), lambda i: (i, 0)),
            pl.BlockSpec((blk, F), lambda i: (i, 0)),
            pl.BlockSpec((blk, F), lambda i: (i, 0)),
        ],
        out_specs=pl.BlockSpec((blk, WMW), lambda i: (i, 0)),
        out_shape=jax.ShapeDtypeStruct((N, WMW), jnp.float32),
    )(s, W1, b1.reshape(1, F), W2, b2.reshape(1, F3), vtx, vty, vtz)


def _wmat_body(en_ref, diff_ref, wr_ref, br_ref, out_ref):
    en = en_ref[0, 0, :]
    nk = (lax.broadcasted_iota(jnp.int32, (1, NRBF), 1) + 1).astype(
        jnp.float32) * (np.pi / CUT)
    rbf = jnp.sin(en[:, None] * nk) / en[:, None]
    wm = jnp.dot(rbf, wr_ref[...], preferred_element_type=jnp.float32) + br_ref[...]
    fcut = 0.5 * (jnp.cos(en * (np.pi / CUT)) + 1.0)
    wm = wm * fcut[:, None]
    out_ref[:, :F3] = wm
    blk = en.shape[0]
    u3 = diff_ref[...] * (1.0 / en)[:, None]
    out_ref[:, F3:] = jnp.concatenate(
        [u3, jnp.zeros((blk, F - 3), jnp.float32)], axis=1)


def _compute_wmat(edges_norm, edges_diff, Wr, br):
    blk = 2000
    return pl.pallas_call(
        _wmat_body,
        grid=(E // blk,),
        in_specs=[
            pl.BlockSpec((1, 1, blk), lambda i: (i, 0, 0)),
            pl.BlockSpec((blk, 3), lambda i: (i, 0)),
            pl.BlockSpec((NRBF, F3), lambda i: (0, 0)),
            pl.BlockSpec((1, F3), lambda i: (0, 0)),
        ],
        out_specs=pl.BlockSpec((blk, WM2), lambda i: (i, 0)),
        out_shape=jax.ShapeDtypeStruct((E, WM2), jnp.float32),
    )(edges_norm.reshape(E // blk, 1, blk), edges_diff, Wr, br.reshape(1, F3))


def _sc_edges_body(phiv, wmat, it_h, jt_h, out,
                   slab, cbufa, jcbufa, cbufb, jcbufb, sel,
                   pva, pvb, wma, wmb, iloca, ilocb,
                   jidxa, jidxb, eidxa, eidxb, sema, semb, semf,
                   semca, semcb):
    c = lax.axis_index("c")
    s = lax.axis_index("s")
    wid = s * NC + c
    iota16 = lax.broadcasted_iota(jnp.int32, (16,), 0)
    zero16 = jnp.zeros((16,), jnp.float32)

    bufs = ((pva, wma, iloca, jidxa, eidxa, sema),
            (pvb, wmb, ilocb, jidxb, eidxb, semb))

    def _prep(b, cb, lo, bset, cbuf, jcbuf):
        pv, wm, iloc_r, jidx, eidx, sem = bset
        ecand = sel[pl.ds(b, 16)]
        valid = ecand >= 0
        e16 = jnp.where(valid, ecand, 0)
        j16 = plsc.load_gather(jcbuf, [e16])
        i16 = plsc.load_gather(cbuf, [e16])
        jidx[...] = j16
        eidx[...] = e16 + cb
        iloc_r[...] = jnp.where(valid, i16 - lo, DUMMY)

        @pl.when(jnp.any(valid))
        def _():
            pltpu.async_copy(phiv.at[jidx], pv, sem)
            pltpu.async_copy(wmat.at[eidx], wm, sem)

    def _compute(bset):
        pv, wm, iloc_r, jidx, eidx, sem = bset

        @pl.when(jnp.any(iloc_r[...] != DUMMY))
        def _go():
            pltpu.make_async_copy(phiv.at[pl.ds(0, K)], pv, sem).wait()
            pltpu.make_async_copy(wmat.at[pl.ds(0, K)], wm, sem).wait()

            @plsc.parallel_loop(0, K, 1, unroll=2)
            def _edge(q):
                qv = jnp.full((16,), q, jnp.int32)
                rb = plsc.load_gather(iloc_r, [qv])
                uq = [plsc.load_gather(wm, [qv, jnp.full((16,), F3 + d, jnp.int32)])
                      for d in range(3)]
                for t in range(8):
                    s2 = (pv[q, pl.ds(F + 16 * t, 16)]
                          * wm[q, pl.ds(F + 16 * t, 16)])
                    plsc.addupdate_scatter(slab, [rb, F3 + 16 * t + iota16], s2)
                for t in range(8):
                    fidx = 16 * t + iota16
                    s1t = pv[q, pl.ds(16 * t, 16)] * wm[q, pl.ds(16 * t, 16)]
                    t3 = (pv[q, pl.ds(2 * F + 16 * t, 16)]
                          * wm[q, pl.ds(2 * F + 16 * t, 16)])
                    for d in range(3):
                        val = (s1t * pv[q, pl.ds(F3 + d * F + 16 * t, 16)]
                               + t3 * uq[d])
                        plsc.addupdate_scatter(slab, [rb, 3 * fidx + d], val)

    def _drain(bset):
        pv, wm, iloc_r, _, _, sem = bset

        @pl.when(jnp.any(iloc_r[...] != DUMMY))
        def _():
            pltpu.make_async_copy(phiv.at[pl.ds(0, K)], pv, sem).wait()
            pltpu.make_async_copy(wmat.at[pl.ds(0, K)], wm, sem).wait()

    def _pass(p, _0):
        lo = (wid * P + p) * RT

        @plsc.parallel_loop(0, SLAB_ROWS * (CW // 16), 1, unroll=4)
        def _zs(t):
            slab[t // (CW // 16), pl.ds((t % (CW // 16)) * 16, 16)] = zero16

        def _stage(ch, cbuf, jcbuf, semc):
            cb = jnp.minimum(ch, E // CE - 1) * CE
            pltpu.async_copy(it_h.at[pl.ds(cb, CE)], cbuf, semc)
            pltpu.async_copy(jt_h.at[pl.ds(cb, CE)], jcbuf, semc)

        def _stage_wait(cbuf, jcbuf, semc):
            pltpu.make_async_copy(it_h.at[pl.ds(0, CE)], cbuf, semc).wait()
            pltpu.make_async_copy(jt_h.at[pl.ds(0, CE)], jcbuf, semc).wait()

        def _one_chunk(ch, cbuf, jcbuf):
            cb = ch * CE

            @plsc.parallel_loop(0, CE // 16, 1, unroll=2,
                                carry=jnp.zeros((16,), jnp.int32))
            def tot(t, tot_c):
                ii = cbuf[pl.ds(t * 16, 16)]
                m = (ii >= lo) & (ii < lo + RT)
                pref = plsc.cumsum(m.astype(jnp.int32))
                plsc.store_scatter(sel, [tot_c + pref - 1], t * 16 + iota16, mask=m)
                return tot_c + plsc.all_reduce_population_count(m)
            neg1 = jnp.full((16,), -1, jnp.int32)
            for w in range(4):
                plsc.store_scatter(sel, [tot + 16 * w + iota16], neg1)

            _drain(bufs[0])
            _prep(0, cb, lo, bufs[0], cbuf, jcbuf)

            def _batch_cond(carry):
                return carry[1]

            def _batch(carry):
                base, _ = carry
                _prep(base + K, cb, lo, bufs[1], cbuf, jcbuf)
                _compute(bufs[0])
                _prep(base + 2 * K, cb, lo, bufs[0], cbuf, jcbuf)
                _compute(bufs[1])
                cont = (jnp.any(sel[pl.ds(base + 2 * K, 16)] >= 0)
                        | jnp.any(sel[pl.ds(base + 3 * K, 16)] >= 0))
                return base + 2 * K, cont

            lax.while_loop(_batch_cond, _batch, (jnp.int32(0), jnp.bool_(True)))

        def _chunk2(k, _):
            ch = 2 * k
            _stage_wait(cbufa, jcbufa, semca)
            _stage(ch + 1, cbufb, jcbufb, semcb)
            _one_chunk(ch, cbufa, jcbufa)
            _stage_wait(cbufb, jcbufb, semcb)
            _stage(ch + 2, cbufa, jcbufa, semca)
            _one_chunk(ch + 1, cbufb, jcbufb)
            return 0

        jidxa[...] = jnp.zeros((16,), jnp.int32)
        eidxa[...] = jnp.zeros((16,), jnp.int32)
        iloca[...] = jnp.zeros((16,), jnp.int32)
        pltpu.async_copy(phiv.at[jidxa], pva, sema)
        pltpu.async_copy(wmat.at[eidxa], wma, sema)
        _stage(0, cbufa, jcbufa, semca)
        lax.fori_loop(0, E // CE // 2, _chunk2, 0)
        _stage_wait(cbufa, jcbufa, semca)

        cps = []
        for t in range(RT // 16):
            cps.append(pltpu.async_copy(slab.at[pl.ds(t * 16, 16)],
                                        out.at[pl.ds(lo + t * 16, 16)], semf))
        _drain(bufs[0])
        for cp in cps:
            cp.wait()
        return 0

    lax.fori_loop(0, P, _pass, 0)


def _sc_edges(phiv, wmat, it_, jt_):
    f32, i32 = jnp.float32, jnp.int32
    mesh = plsc.VectorSubcoreMesh(core_axis_name="c", subcore_axis_name="s")
    fn = pl.kernel(
        _sc_edges_body,
        out_type=jax.ShapeDtypeStruct((NOUT, CW), f32),
        mesh=mesh,
        compiler_params=pltpu.CompilerParams(needs_layout_passes=False),
        scratch_types=[
            pltpu.VMEM((SLAB_ROWS, CW), f32),
            pltpu.VMEM((CE,), i32),
            pltpu.VMEM((CE,), i32),
            pltpu.VMEM((CE,), i32),
            pltpu.VMEM((CE,), i32),
            pltpu.VMEM((CE + 80,), i32),
            pltpu.VMEM((K, WMW), f32),
            pltpu.VMEM((K, WMW), f32),
            pltpu.VMEM((K, WM2), f32),
            pltpu.VMEM((K, WM2), f32),
            pltpu.VMEM((K,), i32),
            pltpu.VMEM((K,), i32),
            pltpu.VMEM((K,), i32),
            pltpu.VMEM((K,), i32),
            pltpu.VMEM((K,), i32),
            pltpu.VMEM((K,), i32),
            pltpu.SemaphoreType.DMA,
            pltpu.SemaphoreType.DMA,
            pltpu.SemaphoreType.DMA,
            pltpu.SemaphoreType.DMA,
            pltpu.SemaphoreType.DMA,
        ],
    )
    return fn(phiv, wmat, it_, jt_)


def kernel(v, s, edges_indices, edges_diff, edges_norm, W1, b1, W2, b2, Wr, br):
    vt = v.transpose(2, 0, 1)
    phiv = _compute_phiv(s, W1, b1, W2, b2, vt[0], vt[1], vt[2])
    wmat = _compute_wmat(edges_norm, edges_diff, Wr, br)
    it_ = edges_indices[:, 0]
    jt_ = edges_indices[:, 1]
    out = _sc_edges(phiv, wmat, it_, jt_)
    dv = out[:N, :F3].reshape(N, F, 3)
    ds = out[:N, F3:]
    return (dv, ds)


# speedup vs baseline: 1.3508x; 1.3508x over previous
import functools

import jax
import jax.numpy as jnp
import numpy as np
from jax import lax
from jax.experimental import pallas as pl
from jax.experimental.pallas import tpu as pltpu
from jax.experimental.pallas import tpu_sc as plsc

F = 128
F3 = 384
NRBF = 20
CUT = 5.0
N = 10000
E = 160000

NC, NS = 2, 16
P = 3
NW = NC * NS
RT = 112
NOUT = P * NW * RT
SLAB_ROWS = 128
DUMMY = RT
CE = 2000
K = 16
CW = F3 + F
WMW = F3 + 3 * F
WM2 = F3 + F


def _phiv_body(s_ref, w1_ref, b1_ref, w2_ref, b2_ref, vx_ref, vy_ref, vz_ref,
               out_ref):
    x = s_ref[...]
    h = jnp.dot(x, w1_ref[...], preferred_element_type=jnp.float32) + b1_ref[...]
    h = h * jax.nn.sigmoid(h)
    out_ref[:, :F3] = (
        jnp.dot(h, w2_ref[...], preferred_element_type=jnp.float32) + b2_ref[...]
    )
    out_ref[:, F3:F3 + F] = vx_ref[...]
    out_ref[:, F3 + F:F3 + 2 * F] = vy_ref[...]
    out_ref[:, F3 + 2 * F:] = vz_ref[...]


def _compute_phiv(s, W1, b1, W2, b2, vtx, vty, vtz):
    blk = 2000
    return pl.pallas_call(
        _phiv_body,
        grid=(N // blk,),
        in_specs=[
            pl.BlockSpec((blk, F), lambda i: (i, 0)),
            pl.BlockSpec((F, F), lambda i: (0, 0)),
            pl.BlockSpec((1, F), lambda i: (0, 0)),
            pl.BlockSpec((F, F3), lambda i: (0, 0)),
            pl.BlockSpec((1, F3), lambda i: (0, 0)),
            pl.BlockSpec((blk, F), lambda i: (i, 0)),
            pl.BlockSpec((blk, F), lambda i: (i, 0)),
            pl.BlockSpec((blk, F), lambda i: (i, 0)),
        ],
        out_specs=pl.BlockSpec((blk, WMW), lambda i: (i, 0)),
        out_shape=jax.ShapeDtypeStruct((N, WMW), jnp.float32),
    )(s, W1, b1.reshape(1, F), W2, b2.reshape(1, F3), vtx, vty, vtz)


def _wmat_body(en_ref, diff_ref, wr_ref, br_ref, out_ref):
    en = en_ref[0, 0, :]
    nk = (lax.broadcasted_iota(jnp.int32, (1, NRBF), 1) + 1).astype(
        jnp.float32) * (np.pi / CUT)
    rbf = jnp.sin(en[:, None] * nk) / en[:, None]
    wm = jnp.dot(rbf, wr_ref[...], preferred_element_type=jnp.float32) + br_ref[...]
    fcut = 0.5 * (jnp.cos(en * (np.pi / CUT)) + 1.0)
    wm = wm * fcut[:, None]
    out_ref[:, :F3] = wm
    blk = en.shape[0]
    u3 = diff_ref[...] * (1.0 / en)[:, None]
    out_ref[:, F3:] = jnp.concatenate(
        [u3, jnp.zeros((blk, F - 3), jnp.float32)], axis=1)


def _compute_wmat(edges_norm, edges_diff, Wr, br):
    blk = 2000
    return pl.pallas_call(
        _wmat_body,
        grid=(E // blk,),
        in_specs=[
            pl.BlockSpec((1, 1, blk), lambda i: (i, 0, 0)),
            pl.BlockSpec((blk, 3), lambda i: (i, 0)),
            pl.BlockSpec((NRBF, F3), lambda i: (0, 0)),
            pl.BlockSpec((1, F3), lambda i: (0, 0)),
        ],
        out_specs=pl.BlockSpec((blk, WM2), lambda i: (i, 0)),
        out_shape=jax.ShapeDtypeStruct((E, WM2), jnp.float32),
    )(edges_norm.reshape(E // blk, 1, blk), edges_diff, Wr, br.reshape(1, F3))


def _sc_edges_body(phiv, wmat, it_h, jt_h, out,
                   slab, cbufa, jcbufa, cbufb, jcbufb, sel,
                   pva, pvb, wma, wmb, iloca, ilocb,
                   jidxa, jidxb, eidxa, eidxb, sema, semb, semf,
                   semca, semcb):
    c = lax.axis_index("c")
    s = lax.axis_index("s")
    wid = s * NC + c
    iota16 = lax.broadcasted_iota(jnp.int32, (16,), 0)
    zero16 = jnp.zeros((16,), jnp.float32)

    bufs = ((pva, wma, iloca, jidxa, eidxa, sema),
            (pvb, wmb, ilocb, jidxb, eidxb, semb))

    def _prep(b, cb, lo, bset, cbuf, jcbuf):
        pv, wm, iloc_r, jidx, eidx, sem = bset
        ecand = sel[pl.ds(b, 16)]
        valid = ecand >= 0
        e16 = jnp.where(valid, ecand, 0)
        j16 = plsc.load_gather(jcbuf, [e16])
        i16 = plsc.load_gather(cbuf, [e16])
        jidx[...] = j16
        eidx[...] = e16 + cb
        iloc_r[...] = jnp.where(valid, i16 - lo, DUMMY)

        @pl.when(jnp.any(valid))
        def _():
            pltpu.async_copy(phiv.at[jidx], pv, sem)
            pltpu.async_copy(wmat.at[eidx], wm, sem)

    def _compute(bset):
        pv, wm, iloc_r, jidx, eidx, sem = bset

        @pl.when(jnp.any(iloc_r[...] != DUMMY))
        def _go():
            pltpu.make_async_copy(phiv.at[pl.ds(0, K)], pv, sem).wait()
            pltpu.make_async_copy(wmat.at[pl.ds(0, K)], wm, sem).wait()

            @plsc.parallel_loop(0, K, 1, unroll=1)
            def _edge(q):
                qv = jnp.full((16,), q, jnp.int32)
                rb = plsc.load_gather(iloc_r, [qv])
                uq = [plsc.load_gather(wm, [qv, jnp.full((16,), F3 + d, jnp.int32)])
                      for d in range(3)]
                for t in range(8):
                    s2 = (pv[q, pl.ds(F + 16 * t, 16)]
                          * wm[q, pl.ds(F + 16 * t, 16)])
                    plsc.addupdate_scatter(slab, [rb, F3 + 16 * t + iota16], s2)
                for t in range(8):
                    fidx = 16 * t + iota16
                    s1t = pv[q, pl.ds(16 * t, 16)] * wm[q, pl.ds(16 * t, 16)]
                    t3 = (pv[q, pl.ds(2 * F + 16 * t, 16)]
                          * wm[q, pl.ds(2 * F + 16 * t, 16)])
                    for d in range(3):
                        val = (s1t * pv[q, pl.ds(F3 + d * F + 16 * t, 16)]
                               + t3 * uq[d])
                        plsc.addupdate_scatter(slab, [rb, 3 * fidx + d], val)

    def _drain(bset):
        pv, wm, iloc_r, _, _, sem = bset

        @pl.when(jnp.any(iloc_r[...] != DUMMY))
        def _():
            pltpu.make_async_copy(phiv.at[pl.ds(0, K)], pv, sem).wait()
            pltpu.make_async_copy(wmat.at[pl.ds(0, K)], wm, sem).wait()

    def _pass(p, _0):
        lo = (wid * P + p) * RT

        @plsc.parallel_loop(0, SLAB_ROWS * (CW // 16), 1, unroll=4)
        def _zs(t):
            slab[t // (CW // 16), pl.ds((t % (CW // 16)) * 16, 16)] = zero16

        def _stage(ch, cbuf, jcbuf, semc):
            cb = jnp.minimum(ch, E // CE - 1) * CE
            pltpu.async_copy(it_h.at[pl.ds(cb, CE)], cbuf, semc)
            pltpu.async_copy(jt_h.at[pl.ds(cb, CE)], jcbuf, semc)

        def _stage_wait(cbuf, jcbuf, semc):
            pltpu.make_async_copy(it_h.at[pl.ds(0, CE)], cbuf, semc).wait()
            pltpu.make_async_copy(jt_h.at[pl.ds(0, CE)], jcbuf, semc).wait()

        def _one_chunk(ch, cbuf, jcbuf):
            cb = ch * CE

            @plsc.parallel_loop(0, CE // 16, 1, unroll=2,
                                carry=jnp.zeros((16,), jnp.int32))
            def tot(t, tot_c):
                ii = cbuf[pl.ds(t * 16, 16)]
                m = (ii >= lo) & (ii < lo + RT)
                pref = plsc.cumsum(m.astype(jnp.int32))
                plsc.store_scatter(sel, [tot_c + pref - 1], t * 16 + iota16, mask=m)
                return tot_c + plsc.all_reduce_population_count(m)
            neg1 = jnp.full((16,), -1, jnp.int32)
            for w in range(4):
                plsc.store_scatter(sel, [tot + 16 * w + iota16], neg1)

            _drain(bufs[0])
            _prep(0, cb, lo, bufs[0], cbuf, jcbuf)

            def _batch_cond(carry):
                return carry[1]

            def _batch(carry):
                base, _ = carry
                _prep(base + K, cb, lo, bufs[1], cbuf, jcbuf)
                _compute(bufs[0])
                _prep(base + 2 * K, cb, lo, bufs[0], cbuf, jcbuf)
                _compute(bufs[1])
                cont = (jnp.any(sel[pl.ds(base + 2 * K, 16)] >= 0)
                        | jnp.any(sel[pl.ds(base + 3 * K, 16)] >= 0))
                return base + 2 * K, cont

            lax.while_loop(_batch_cond, _batch, (jnp.int32(0), jnp.bool_(True)))

        def _chunk2(k, _):
            ch = 2 * k
            _stage_wait(cbufa, jcbufa, semca)
            _stage(ch + 1, cbufb, jcbufb, semcb)
            _one_chunk(ch, cbufa, jcbufa)
            _stage_wait(cbufb, jcbufb, semcb)
            _stage(ch + 2, cbufa, jcbufa, semca)
            _one_chunk(ch + 1, cbufb, jcbufb)
            return 0

        jidxa[...] = jnp.zeros((16,), jnp.int32)
        eidxa[...] = jnp.zeros((16,), jnp.int32)
        iloca[...] = jnp.zeros((16,), jnp.int32)
        pltpu.async_copy(phiv.at[jidxa], pva, sema)
        pltpu.async_copy(wmat.at[eidxa], wma, sema)
        _stage(0, cbufa, jcbufa, semca)
        lax.fori_loop(0, E // CE // 2, _chunk2, 0)
        _stage_wait(cbufa, jcbufa, semca)

        cps = []
        for t in range(RT // 16):
            cps.append(pltpu.async_copy(slab.at[pl.ds(t * 16, 16)],
                                        out.at[pl.ds(lo + t * 16, 16)], semf))
        _drain(bufs[0])
        for cp in cps:
            cp.wait()
        return 0

    lax.fori_loop(0, P, _pass, 0)


def _sc_edges(phiv, wmat, it_, jt_):
    f32, i32 = jnp.float32, jnp.int32
    mesh = plsc.VectorSubcoreMesh(core_axis_name="c", subcore_axis_name="s")
    fn = pl.kernel(
        _sc_edges_body,
        out_type=jax.ShapeDtypeStruct((NOUT, CW), f32),
        mesh=mesh,
        compiler_params=pltpu.CompilerParams(needs_layout_passes=False),
        scratch_types=[
            pltpu.VMEM((SLAB_ROWS, CW), f32),
            pltpu.VMEM((CE,), i32),
            pltpu.VMEM((CE,), i32),
            pltpu.VMEM((CE,), i32),
            pltpu.VMEM((CE,), i32),
            pltpu.VMEM((CE + 80,), i32),
            pltpu.VMEM((K, WMW), f32),
            pltpu.VMEM((K, WMW), f32),
            pltpu.VMEM((K, WM2), f32),
            pltpu.VMEM((K, WM2), f32),
            pltpu.VMEM((K,), i32),
            pltpu.VMEM((K,), i32),
            pltpu.VMEM((K,), i32),
            pltpu.VMEM((K,), i32),
            pltpu.VMEM((K,), i32),
            pltpu.VMEM((K,), i32),
            pltpu.SemaphoreType.DMA,
            pltpu.SemaphoreType.DMA,
            pltpu.SemaphoreType.DMA,
            pltpu.SemaphoreType.DMA,
            pltpu.SemaphoreType.DMA,
        ],
    )
    return fn(phiv, wmat, it_, jt_)


def kernel(v, s, edges_indices, edges_diff, edges_norm, W1, b1, W2, b2, Wr, br):
    vt = v.transpose(2, 0, 1)
    phiv = _compute_phiv(s, W1, b1, W2, b2, vt[0], vt[1], vt[2])
    wmat = _compute_wmat(edges_norm, edges_diff, Wr, br)
    it_ = edges_indices[:, 0]
    jt_ = edges_indices[:, 1]
    out = _sc_edges(phiv, wmat, it_, jt_)
    dv = out[:N, :F3].reshape(N, F, 3)
    ds = out[:N, F3:]
    return (dv, ds)


# speedup vs baseline: 1.6220x; 1.2007x over previous
import functools

import jax
import jax.numpy as jnp
import numpy as np
from jax import lax
from jax.experimental import pallas as pl
from jax.experimental.pallas import tpu as pltpu
from jax.experimental.pallas import tpu_sc as plsc

F = 128
F3 = 384
NRBF = 20
CUT = 5.0
N = 10000
E = 160000

NC, NS = 2, 16
P = 3
NW = NC * NS
RT = 112
NOUT = P * NW * RT
SLAB_ROWS = 128
DUMMY = RT
CE = 4000
K = 16
CW = F3 + F
WMW = F3 + 3 * F
WM2 = F3 + F


def _phiv_body(s_ref, w1_ref, b1_ref, w2_ref, b2_ref, vx_ref, vy_ref, vz_ref,
               out_ref):
    x = s_ref[...]
    h = jnp.dot(x, w1_ref[...], preferred_element_type=jnp.float32) + b1_ref[...]
    h = h * jax.nn.sigmoid(h)
    out_ref[:, :F3] = (
        jnp.dot(h, w2_ref[...], preferred_element_type=jnp.float32) + b2_ref[...]
    )
    out_ref[:, F3:F3 + F] = vx_ref[...]
    out_ref[:, F3 + F:F3 + 2 * F] = vy_ref[...]
    out_ref[:, F3 + 2 * F:] = vz_ref[...]


def _compute_phiv(s, W1, b1, W2, b2, vtx, vty, vtz):
    blk = 2000
    return pl.pallas_call(
        _phiv_body,
        grid=(N // blk,),
        in_specs=[
            pl.BlockSpec((blk, F), lambda i: (i, 0)),
            pl.BlockSpec((F, F), lambda i: (0, 0)),
            pl.BlockSpec((1, F), lambda i: (0, 0)),
            pl.BlockSpec((F, F3), lambda i: (0, 0)),
            pl.BlockSpec((1, F3), lambda i: (0, 0)),
            pl.BlockSpec((blk, F), lambda i: (i, 0)),
            pl.BlockSpec((blk, F), lambda i: (i, 0)),
            pl.BlockSpec((blk, F), lambda i: (i, 0)),
        ],
        out_specs=pl.BlockSpec((blk, WMW), lambda i: (i, 0)),
        out_shape=jax.ShapeDtypeStruct((N, WMW), jnp.float32),
    )(s, W1, b1.reshape(1, F), W2, b2.reshape(1, F3), vtx, vty, vtz)


def _wmat_body(en_ref, diff_ref, wr_ref, br_ref, out_ref):
    en = en_ref[0, 0, :]
    nk = (lax.broadcasted_iota(jnp.int32, (1, NRBF), 1) + 1).astype(
        jnp.float32) * (np.pi / CUT)
    rbf = jnp.sin(en[:, None] * nk) / en[:, None]
    wm = jnp.dot(rbf, wr_ref[...], preferred_element_type=jnp.float32) + br_ref[...]
    fcut = 0.5 * (jnp.cos(en * (np.pi / CUT)) + 1.0)
    wm = wm * fcut[:, None]
    out_ref[:, :F3] = wm
    blk = en.shape[0]
    u3 = diff_ref[...] * (1.0 / en)[:, None]
    out_ref[:, F3:] = jnp.concatenate(
        [u3, jnp.zeros((blk, F - 3), jnp.float32)], axis=1)


def _compute_wmat(edges_norm, edges_diff, Wr, br):
    blk = 2000
    return pl.pallas_call(
        _wmat_body,
        grid=(E // blk,),
        in_specs=[
            pl.BlockSpec((1, 1, blk), lambda i: (i, 0, 0)),
            pl.BlockSpec((blk, 3), lambda i: (i, 0)),
            pl.BlockSpec((NRBF, F3), lambda i: (0, 0)),
            pl.BlockSpec((1, F3), lambda i: (0, 0)),
        ],
        out_specs=pl.BlockSpec((blk, WM2), lambda i: (i, 0)),
        out_shape=jax.ShapeDtypeStruct((E, WM2), jnp.float32),
    )(edges_norm.reshape(E // blk, 1, blk), edges_diff, Wr, br.reshape(1, F3))


def _sc_edges_body(phiv, wmat, it_h, jt_h, out,
                   slab, cbufa, jcbufa, cbufb, jcbufb, sel,
                   pva, pvb, wma, wmb, iloca, ilocb,
                   jidxa, jidxb, eidxa, eidxb, sema, semb, semf,
                   semca, semcb):
    c = lax.axis_index("c")
    s = lax.axis_index("s")
    wid = s * NC + c
    iota16 = lax.broadcasted_iota(jnp.int32, (16,), 0)
    zero16 = jnp.zeros((16,), jnp.float32)

    bufs = ((pva, wma, iloca, jidxa, eidxa, sema),
            (pvb, wmb, ilocb, jidxb, eidxb, semb))

    def _prep(b, cb, lo, bset, cbuf, jcbuf):
        pv, wm, iloc_r, jidx, eidx, sem = bset
        ecand = sel[pl.ds(b, 16)]
        valid = ecand >= 0
        e16 = jnp.where(valid, ecand, 0)
        j16 = plsc.load_gather(jcbuf, [e16])
        i16 = plsc.load_gather(cbuf, [e16])
        jidx[...] = j16
        eidx[...] = e16 + cb
        iloc_r[...] = jnp.where(valid, i16 - lo, DUMMY)

        @pl.when(jnp.any(valid))
        def _():
            pltpu.async_copy(phiv.at[jidx], pv, sem)
            pltpu.async_copy(wmat.at[eidx], wm, sem)

    def _compute(bset):
        pv, wm, iloc_r, jidx, eidx, sem = bset

        @pl.when(jnp.any(iloc_r[...] != DUMMY))
        def _go():
            pltpu.make_async_copy(phiv.at[pl.ds(0, K)], pv, sem).wait()
            pltpu.make_async_copy(wmat.at[pl.ds(0, K)], wm, sem).wait()

            @plsc.parallel_loop(0, K, 1, unroll=1)
            def _edge(q):
                qv = jnp.full((16,), q, jnp.int32)
                rb = plsc.load_gather(iloc_r, [qv])
                uq = [plsc.load_gather(wm, [qv, jnp.full((16,), F3 + d, jnp.int32)])
                      for d in range(3)]
                for t in range(8):
                    s2 = (pv[q, pl.ds(F + 16 * t, 16)]
                          * wm[q, pl.ds(F + 16 * t, 16)])
                    plsc.addupdate_scatter(slab, [rb, F3 + 16 * t + iota16], s2)
                for t in range(8):
                    fidx = 16 * t + iota16
                    s1t = pv[q, pl.ds(16 * t, 16)] * wm[q, pl.ds(16 * t, 16)]
                    t3 = (pv[q, pl.ds(2 * F + 16 * t, 16)]
                          * wm[q, pl.ds(2 * F + 16 * t, 16)])
                    for d in range(3):
                        val = (s1t * pv[q, pl.ds(F3 + d * F + 16 * t, 16)]
                               + t3 * uq[d])
                        plsc.addupdate_scatter(slab, [rb, 3 * fidx + d], val)

    def _drain(bset):
        pv, wm, iloc_r, _, _, sem = bset

        @pl.when(jnp.any(iloc_r[...] != DUMMY))
        def _():
            pltpu.make_async_copy(phiv.at[pl.ds(0, K)], pv, sem).wait()
            pltpu.make_async_copy(wmat.at[pl.ds(0, K)], wm, sem).wait()

    def _pass(p, _0):
        lo = (wid * P + p) * RT

        @plsc.parallel_loop(0, SLAB_ROWS * (CW // 16), 1, unroll=4)
        def _zs(t):
            slab[t // (CW // 16), pl.ds((t % (CW // 16)) * 16, 16)] = zero16

        def _stage(ch, cbuf, jcbuf, semc):
            cb = jnp.minimum(ch, E // CE - 1) * CE
            pltpu.async_copy(it_h.at[pl.ds(cb, CE)], cbuf, semc)
            pltpu.async_copy(jt_h.at[pl.ds(cb, CE)], jcbuf, semc)

        def _stage_wait(cbuf, jcbuf, semc):
            pltpu.make_async_copy(it_h.at[pl.ds(0, CE)], cbuf, semc).wait()
            pltpu.make_async_copy(jt_h.at[pl.ds(0, CE)], jcbuf, semc).wait()

        def _one_chunk(ch, cbuf, jcbuf):
            cb = ch * CE

            @plsc.parallel_loop(0, CE // 16, 1, unroll=2,
                                carry=jnp.zeros((16,), jnp.int32))
            def tot(t, tot_c):
                ii = cbuf[pl.ds(t * 16, 16)]
                m = (ii >= lo) & (ii < lo + RT)
                pref = plsc.cumsum(m.astype(jnp.int32))
                plsc.store_scatter(sel, [tot_c + pref - 1], t * 16 + iota16, mask=m)
                return tot_c + plsc.all_reduce_population_count(m)
            neg1 = jnp.full((16,), -1, jnp.int32)
            for w in range(4):
                plsc.store_scatter(sel, [tot + 16 * w + iota16], neg1)

            _drain(bufs[0])
            _prep(0, cb, lo, bufs[0], cbuf, jcbuf)

            def _batch_cond(carry):
                return carry[1]

            def _batch(carry):
                base, _ = carry
                _prep(base + K, cb, lo, bufs[1], cbuf, jcbuf)
                _compute(bufs[0])
                _prep(base + 2 * K, cb, lo, bufs[0], cbuf, jcbuf)
                _compute(bufs[1])
                cont = (jnp.any(sel[pl.ds(base + 2 * K, 16)] >= 0)
                        | jnp.any(sel[pl.ds(base + 3 * K, 16)] >= 0))
                return base + 2 * K, cont

            lax.while_loop(_batch_cond, _batch, (jnp.int32(0), jnp.bool_(True)))

        def _chunk2(k, _):
            ch = 2 * k
            _stage_wait(cbufa, jcbufa, semca)
            _stage(ch + 1, cbufb, jcbufb, semcb)
            _one_chunk(ch, cbufa, jcbufa)
            _stage_wait(cbufb, jcbufb, semcb)
            _stage(ch + 2, cbufa, jcbufa, semca)
            _one_chunk(ch + 1, cbufb, jcbufb)
            return 0

        jidxa[...] = jnp.zeros((16,), jnp.int32)
        eidxa[...] = jnp.zeros((16,), jnp.int32)
        iloca[...] = jnp.zeros((16,), jnp.int32)
        pltpu.async_copy(phiv.at[jidxa], pva, sema)
        pltpu.async_copy(wmat.at[eidxa], wma, sema)
        _stage(0, cbufa, jcbufa, semca)
        lax.fori_loop(0, E // CE // 2, _chunk2, 0)
        _stage_wait(cbufa, jcbufa, semca)

        cps = []
        for t in range(RT // 16):
            cps.append(pltpu.async_copy(slab.at[pl.ds(t * 16, 16)],
                                        out.at[pl.ds(lo + t * 16, 16)], semf))
        _drain(bufs[0])
        for cp in cps:
            cp.wait()
        return 0

    lax.fori_loop(0, P, _pass, 0)


def _sc_edges(phiv, wmat, it_, jt_):
    f32, i32 = jnp.float32, jnp.int32
    mesh = plsc.VectorSubcoreMesh(core_axis_name="c", subcore_axis_name="s")
    fn = pl.kernel(
        _sc_edges_body,
        out_type=jax.ShapeDtypeStruct((NOUT, CW), f32),
        mesh=mesh,
        compiler_params=pltpu.CompilerParams(needs_layout_passes=False),
        scratch_types=[
            pltpu.VMEM((SLAB_ROWS, CW), f32),
            pltpu.VMEM((CE,), i32),
            pltpu.VMEM((CE,), i32),
            pltpu.VMEM((CE,), i32),
            pltpu.VMEM((CE,), i32),
            pltpu.VMEM((CE + 80,), i32),
            pltpu.VMEM((K, WMW), f32),
            pltpu.VMEM((K, WMW), f32),
            pltpu.VMEM((K, WM2), f32),
            pltpu.VMEM((K, WM2), f32),
            pltpu.VMEM((K,), i32),
            pltpu.VMEM((K,), i32),
            pltpu.VMEM((K,), i32),
            pltpu.VMEM((K,), i32),
            pltpu.VMEM((K,), i32),
            pltpu.VMEM((K,), i32),
            pltpu.SemaphoreType.DMA,
            pltpu.SemaphoreType.DMA,
            pltpu.SemaphoreType.DMA,
            pltpu.SemaphoreType.DMA,
            pltpu.SemaphoreType.DMA,
        ],
    )
    return fn(phiv, wmat, it_, jt_)


def kernel(v, s, edges_indices, edges_diff, edges_norm, W1, b1, W2, b2, Wr, br):
    vt = v.transpose(2, 0, 1)
    phiv = _compute_phiv(s, W1, b1, W2, b2, vt[0], vt[1], vt[2])
    wmat = _compute_wmat(edges_norm, edges_diff, Wr, br)
    it_ = edges_indices[:, 0]
    jt_ = edges_indices[:, 1]
    out = _sc_edges(phiv, wmat, it_, jt_)
    dv = out[:N, :F3].reshape(N, F, 3)
    ds = out[:N, F3:]
    return (dv, ds)


# speedup vs baseline: 1.6455x; 1.0145x over previous
import functools

import jax
import jax.numpy as jnp
import numpy as np
from jax import lax
from jax.experimental import pallas as pl
from jax.experimental.pallas import tpu as pltpu
from jax.experimental.pallas import tpu_sc as plsc

F = 128
F3 = 384
NRBF = 20
CUT = 5.0
N = 10000
E = 160000

NC, NS = 2, 16
P = 3
NW = NC * NS
RT = 112
NOUT = P * NW * RT
SLAB_ROWS = 128
DUMMY = RT
CE = 4000
K = 16
CW = F3 + F
WMW = F3 + 3 * F
WM2 = F3 + F


def _phiv_body(s_ref, w1_ref, b1_ref, w2_ref, b2_ref, vx_ref, vy_ref, vz_ref,
               out_ref):
    x = s_ref[...]
    h = jnp.dot(x, w1_ref[...], preferred_element_type=jnp.float32) + b1_ref[...]
    h = h * jax.nn.sigmoid(h)
    out_ref[:, :F3] = (
        jnp.dot(h, w2_ref[...], preferred_element_type=jnp.float32) + b2_ref[...]
    )
    out_ref[:, F3:F3 + F] = vx_ref[...]
    out_ref[:, F3 + F:F3 + 2 * F] = vy_ref[...]
    out_ref[:, F3 + 2 * F:] = vz_ref[...]


def _compute_phiv(s, W1, b1, W2, b2, vtx, vty, vtz):
    blk = 2000
    return pl.pallas_call(
        _phiv_body,
        grid=(N // blk,),
        in_specs=[
            pl.BlockSpec((blk, F), lambda i: (i, 0)),
            pl.BlockSpec((F, F), lambda i: (0, 0)),
            pl.BlockSpec((1, F), lambda i: (0, 0)),
            pl.BlockSpec((F, F3), lambda i: (0, 0)),
            pl.BlockSpec((1, F3), lambda i: (0, 0)),
            pl.BlockSpec((blk, F), lambda i: (i, 0)),
            pl.BlockSpec((blk, F), lambda i: (i, 0)),
            pl.BlockSpec((blk, F), lambda i: (i, 0)),
        ],
        out_specs=pl.BlockSpec((blk, WMW), lambda i: (i, 0)),
        out_shape=jax.ShapeDtypeStruct((N, WMW), jnp.float32),
    )(s, W1, b1.reshape(1, F), W2, b2.reshape(1, F3), vtx, vty, vtz)


def _wmat_body(en_ref, diff_ref, wr_ref, br_ref, out_ref):
    en = en_ref[0, 0, :]
    nk = (lax.broadcasted_iota(jnp.int32, (1, NRBF), 1) + 1).astype(
        jnp.float32) * (np.pi / CUT)
    inv = 1.0 / en
    rbf = jnp.sin(en[:, None] * nk) * inv[:, None]
    wm = jnp.dot(rbf, wr_ref[...], preferred_element_type=jnp.float32) + br_ref[...]
    fcut = 0.5 * (jnp.cos(en * (np.pi / CUT)) + 1.0)
    wm = wm * fcut[:, None]
    out_ref[:, :F3] = wm
    blk = en.shape[0]
    u3 = diff_ref[...] * inv[:, None]
    out_ref[:, F3:] = jnp.concatenate(
        [u3, jnp.zeros((blk, F - 3), jnp.float32)], axis=1)


def _compute_wmat(edges_norm, edges_diff, Wr, br):
    blk = 2000
    return pl.pallas_call(
        _wmat_body,
        grid=(E // blk,),
        in_specs=[
            pl.BlockSpec((1, 1, blk), lambda i: (i, 0, 0)),
            pl.BlockSpec((blk, 3), lambda i: (i, 0)),
            pl.BlockSpec((NRBF, F3), lambda i: (0, 0)),
            pl.BlockSpec((1, F3), lambda i: (0, 0)),
        ],
        out_specs=pl.BlockSpec((blk, WM2), lambda i: (i, 0)),
        out_shape=jax.ShapeDtypeStruct((E, WM2), jnp.float32),
    )(edges_norm.reshape(E // blk, 1, blk), edges_diff, Wr, br.reshape(1, F3))


def _sc_edges_body(phiv, wmat, it_h, jt_h, out,
                   slab, cbufa, jcbufa, cbufb, jcbufb, sel,
                   pva, pvb, wma, wmb, iloca, ilocb,
                   jidxa, jidxb, eidxa, eidxb, sema, semb, semf,
                   semca, semcb):
    c = lax.axis_index("c")
    s = lax.axis_index("s")
    wid = s * NC + c
    iota16 = lax.broadcasted_iota(jnp.int32, (16,), 0)
    zero16 = jnp.zeros((16,), jnp.float32)

    bufs = ((pva, wma, iloca, jidxa, eidxa, sema),
            (pvb, wmb, ilocb, jidxb, eidxb, semb))

    def _prep(b, cb, lo, bset, cbuf, jcbuf):
        pv, wm, iloc_r, jidx, eidx, sem = bset
        ecand = sel[pl.ds(b, 16)]
        valid = ecand >= 0
        e16 = jnp.where(valid, ecand, 0)
        j16 = plsc.load_gather(jcbuf, [e16])
        i16 = plsc.load_gather(cbuf, [e16])
        jidx[...] = j16
        eidx[...] = e16 + cb
        iloc_r[...] = jnp.where(valid, i16 - lo, DUMMY)

        @pl.when(jnp.any(valid))
        def _():
            pltpu.async_copy(phiv.at[jidx], pv, sem)
            pltpu.async_copy(wmat.at[eidx], wm, sem)

    def _compute(bset):
        pv, wm, iloc_r, jidx, eidx, sem = bset

        @pl.when(jnp.any(iloc_r[...] != DUMMY))
        def _go():
            pltpu.make_async_copy(phiv.at[pl.ds(0, K)], pv, sem).wait()
            pltpu.make_async_copy(wmat.at[pl.ds(0, K)], wm, sem).wait()

            @plsc.parallel_loop(0, K, 1, unroll=1)
            def _edge(q):
                qv = jnp.full((16,), q, jnp.int32)
                rb = plsc.load_gather(iloc_r, [qv])
                uq = [plsc.load_gather(wm, [qv, jnp.full((16,), F3 + d, jnp.int32)])
                      for d in range(3)]
                for t in range(8):
                    s2 = (pv[q, pl.ds(F + 16 * t, 16)]
                          * wm[q, pl.ds(F + 16 * t, 16)])
                    plsc.addupdate_scatter(slab, [rb, F3 + 16 * t + iota16], s2)
                for t in range(8):
                    fidx = 16 * t + iota16
                    s1t = pv[q, pl.ds(16 * t, 16)] * wm[q, pl.ds(16 * t, 16)]
                    t3 = (pv[q, pl.ds(2 * F + 16 * t, 16)]
                          * wm[q, pl.ds(2 * F + 16 * t, 16)])
                    for d in range(3):
                        val = (s1t * pv[q, pl.ds(F3 + d * F + 16 * t, 16)]
                               + t3 * uq[d])
                        plsc.addupdate_scatter(slab, [rb, 3 * fidx + d], val)

    def _drain(bset):
        pv, wm, iloc_r, _, _, sem = bset

        @pl.when(jnp.any(iloc_r[...] != DUMMY))
        def _():
            pltpu.make_async_copy(phiv.at[pl.ds(0, K)], pv, sem).wait()
            pltpu.make_async_copy(wmat.at[pl.ds(0, K)], wm, sem).wait()

    def _pass(p, _0):
        lo = (wid * P + p) * RT

        @plsc.parallel_loop(0, SLAB_ROWS * (CW // 16), 1, unroll=4)
        def _zs(t):
            slab[t // (CW // 16), pl.ds((t % (CW // 16)) * 16, 16)] = zero16

        def _stage(ch, cbuf, jcbuf, semc):
            cb = jnp.minimum(ch, E // CE - 1) * CE
            pltpu.async_copy(it_h.at[pl.ds(cb, CE)], cbuf, semc)
            pltpu.async_copy(jt_h.at[pl.ds(cb, CE)], jcbuf, semc)

        def _stage_wait(cbuf, jcbuf, semc):
            pltpu.make_async_copy(it_h.at[pl.ds(0, CE)], cbuf, semc).wait()
            pltpu.make_async_copy(jt_h.at[pl.ds(0, CE)], jcbuf, semc).wait()

        def _one_chunk(ch, cbuf, jcbuf):
            cb = ch * CE

            @plsc.parallel_loop(0, CE // 16, 1, unroll=2,
                                carry=jnp.zeros((16,), jnp.int32))
            def tot(t, tot_c):
                ii = cbuf[pl.ds(t * 16, 16)]
                m = (ii >= lo) & (ii < lo + RT)
                pref = plsc.cumsum(m.astype(jnp.int32))
                plsc.store_scatter(sel, [tot_c + pref - 1], t * 16 + iota16, mask=m)
                return tot_c + plsc.all_reduce_population_count(m)
            neg1 = jnp.full((16,), -1, jnp.int32)
            for w in range(4):
                plsc.store_scatter(sel, [tot + 16 * w + iota16], neg1)

            _drain(bufs[0])
            _prep(0, cb, lo, bufs[0], cbuf, jcbuf)

            def _batch_cond(carry):
                return carry[1]

            def _batch(carry):
                base, _ = carry
                _prep(base + K, cb, lo, bufs[1], cbuf, jcbuf)
                _compute(bufs[0])
                _prep(base + 2 * K, cb, lo, bufs[0], cbuf, jcbuf)
                _compute(bufs[1])
                cont = (jnp.any(sel[pl.ds(base + 2 * K, 16)] >= 0)
                        | jnp.any(sel[pl.ds(base + 3 * K, 16)] >= 0))
                return base + 2 * K, cont

            lax.while_loop(_batch_cond, _batch, (jnp.int32(0), jnp.bool_(True)))

        def _chunk2(k, _):
            ch = 2 * k
            _stage_wait(cbufa, jcbufa, semca)
            _stage(ch + 1, cbufb, jcbufb, semcb)
            _one_chunk(ch, cbufa, jcbufa)
            _stage_wait(cbufb, jcbufb, semcb)
            _stage(ch + 2, cbufa, jcbufa, semca)
            _one_chunk(ch + 1, cbufb, jcbufb)
            return 0

        jidxa[...] = jnp.zeros((16,), jnp.int32)
        eidxa[...] = jnp.zeros((16,), jnp.int32)
        iloca[...] = jnp.zeros((16,), jnp.int32)
        pltpu.async_copy(phiv.at[jidxa], pva, sema)
        pltpu.async_copy(wmat.at[eidxa], wma, sema)
        _stage(0, cbufa, jcbufa, semca)
        lax.fori_loop(0, E // CE // 2, _chunk2, 0)
        _stage_wait(cbufa, jcbufa, semca)

        cps = []
        for t in range(RT // 16):
            cps.append(pltpu.async_copy(slab.at[pl.ds(t * 16, 16)],
                                        out.at[pl.ds(lo + t * 16, 16)], semf))
        _drain(bufs[0])
        for cp in cps:
            cp.wait()
        return 0

    lax.fori_loop(0, P, _pass, 0)


def _sc_edges(phiv, wmat, it_, jt_):
    f32, i32 = jnp.float32, jnp.int32
    mesh = plsc.VectorSubcoreMesh(core_axis_name="c", subcore_axis_name="s")
    fn = pl.kernel(
        _sc_edges_body,
        out_type=jax.ShapeDtypeStruct((NOUT, CW), f32),
        mesh=mesh,
        compiler_params=pltpu.CompilerParams(needs_layout_passes=False),
        scratch_types=[
            pltpu.VMEM((SLAB_ROWS, CW), f32),
            pltpu.VMEM((CE,), i32),
            pltpu.VMEM((CE,), i32),
            pltpu.VMEM((CE,), i32),
            pltpu.VMEM((CE,), i32),
            pltpu.VMEM((CE + 80,), i32),
            pltpu.VMEM((K, WMW), f32),
            pltpu.VMEM((K, WMW), f32),
            pltpu.VMEM((K, WM2), f32),
            pltpu.VMEM((K, WM2), f32),
            pltpu.VMEM((K,), i32),
            pltpu.VMEM((K,), i32),
            pltpu.VMEM((K,), i32),
            pltpu.VMEM((K,), i32),
            pltpu.VMEM((K,), i32),
            pltpu.VMEM((K,), i32),
            pltpu.SemaphoreType.DMA,
            pltpu.SemaphoreType.DMA,
            pltpu.SemaphoreType.DMA,
            pltpu.SemaphoreType.DMA,
            pltpu.SemaphoreType.DMA,
        ],
    )
    return fn(phiv, wmat, it_, jt_)


def kernel(v, s, edges_indices, edges_diff, edges_norm, W1, b1, W2, b2, Wr, br):
    vt = v.transpose(2, 0, 1)
    phiv = _compute_phiv(s, W1, b1, W2, b2, vt[0], vt[1], vt[2])
    wmat = _compute_wmat(edges_norm, edges_diff, Wr, br)
    it_ = edges_indices[:, 0]
    jt_ = edges_indices[:, 1]
    out = _sc_edges(phiv, wmat, it_, jt_)
    dv = out[:N, :F3].reshape(N, F, 3)
    ds = out[:N, F3:]
    return (dv, ds)


# speedup vs baseline: 1.7405x; 1.0577x over previous
import functools

import jax
import jax.numpy as jnp
import numpy as np
from jax import lax
from jax.experimental import pallas as pl
from jax.experimental.pallas import tpu as pltpu
from jax.experimental.pallas import tpu_sc as plsc

F = 128
F3 = 384
NRBF = 20
CUT = 5.0
N = 10000
E = 160000

NC, NS = 2, 16
P = 3
NW = NC * NS
RT = 112
NOUT = P * NW * RT
SLAB_ROWS = 128
DUMMY = RT
CE = 4000
K = 16
CW = F3 + F
WMW = F3 + 3 * F
WM2 = F3 + F


def _phiv_body(s_ref, w1_ref, b1_ref, w2_ref, b2_ref, vx_ref, vy_ref, vz_ref,
               out_ref):
    x = s_ref[...]
    h = jnp.dot(x, w1_ref[...], preferred_element_type=jnp.float32) + b1_ref[...]
    h = h * jax.nn.sigmoid(h)
    out_ref[:, :F3] = (
        jnp.dot(h, w2_ref[...], preferred_element_type=jnp.float32) + b2_ref[...]
    )
    out_ref[:, F3:F3 + F] = vx_ref[...]
    out_ref[:, F3 + F:F3 + 2 * F] = vy_ref[...]
    out_ref[:, F3 + 2 * F:] = vz_ref[...]


def _compute_phiv(s, W1, b1, W2, b2, vtx, vty, vtz):
    blk = 2000
    return pl.pallas_call(
        _phiv_body,
        grid=(N // blk,),
        in_specs=[
            pl.BlockSpec((blk, F), lambda i: (i, 0)),
            pl.BlockSpec((F, F), lambda i: (0, 0)),
            pl.BlockSpec((1, F), lambda i: (0, 0)),
            pl.BlockSpec((F, F3), lambda i: (0, 0)),
            pl.BlockSpec((1, F3), lambda i: (0, 0)),
            pl.BlockSpec((blk, F), lambda i: (i, 0)),
            pl.BlockSpec((blk, F), lambda i: (i, 0)),
            pl.BlockSpec((blk, F), lambda i: (i, 0)),
        ],
        out_specs=pl.BlockSpec((blk, WMW), lambda i: (i, 0)),
        out_shape=jax.ShapeDtypeStruct((N, WMW), jnp.float32),
    )(s, W1, b1.reshape(1, F), W2, b2.reshape(1, F3), vtx, vty, vtz)


def _wmat_body(en_ref, diff_ref, wr_ref, br_ref, out_ref):
    en = en_ref[0, 0, :]
    nk = (lax.broadcasted_iota(jnp.int32, (1, NRBF), 1) + 1).astype(
        jnp.float32) * (np.pi / CUT)
    inv = 1.0 / en
    rbf = jnp.sin(en[:, None] * nk) * inv[:, None]
    wm = jnp.dot(rbf, wr_ref[...], preferred_element_type=jnp.float32) + br_ref[...]
    fcut = 0.5 * (jnp.cos(en * (np.pi / CUT)) + 1.0)
    wm = wm * fcut[:, None]
    out_ref[:, :F3] = wm
    blk = en.shape[0]
    u3 = diff_ref[...] * inv[:, None]
    out_ref[:, F3:] = jnp.concatenate(
        [u3, jnp.zeros((blk, F - 3), jnp.float32)], axis=1)


def _compute_wmat(edges_norm, edges_diff, Wr, br):
    blk = 2000
    return pl.pallas_call(
        _wmat_body,
        grid=(E // blk,),
        in_specs=[
            pl.BlockSpec((1, 1, blk), lambda i: (i, 0, 0)),
            pl.BlockSpec((blk, 3), lambda i: (i, 0)),
            pl.BlockSpec((NRBF, F3), lambda i: (0, 0)),
            pl.BlockSpec((1, F3), lambda i: (0, 0)),
        ],
        out_specs=pl.BlockSpec((blk, WM2), lambda i: (i, 0)),
        out_shape=jax.ShapeDtypeStruct((E, WM2), jnp.float32),
    )(edges_norm.reshape(E // blk, 1, blk), edges_diff, Wr, br.reshape(1, F3))


def _sc_edges_body(phiv, wmat, it_h, jt_h, out,
                   slab, cbufa, jcbufa, cbufb, jcbufb, sel,
                   pva, pvb, wma, wmb, iloca, ilocb,
                   jidxa, jidxb, eidxa, eidxb, sema, semb, semf,
                   semca, semcb):
    c = lax.axis_index("c")
    s = lax.axis_index("s")
    wid = s * NC + c
    iota16 = lax.broadcasted_iota(jnp.int32, (16,), 0)
    zero16 = jnp.zeros((16,), jnp.float32)

    bufs = ((pva, wma, iloca, jidxa, eidxa, sema),
            (pvb, wmb, ilocb, jidxb, eidxb, semb))

    def _prep(b, cb, lo, bset, cbuf, jcbuf):
        pv, wm, iloc_r, jidx, eidx, sem = bset
        ecand = sel[pl.ds(b, 16)]
        valid = ecand >= 0
        e16 = jnp.where(valid, ecand, 0)
        j16 = plsc.load_gather(jcbuf, [e16])
        i16 = plsc.load_gather(cbuf, [e16])
        jidx[...] = j16
        eidx[...] = e16 + cb
        iloc_r[...] = jnp.where(valid, i16 - lo, DUMMY)

        @pl.when(jnp.any(valid))
        def _():
            pltpu.async_copy(phiv.at[jidx], pv, sem)
            pltpu.async_copy(wmat.at[eidx], wm, sem)

    def _compute(bset):
        pv, wm, iloc_r, jidx, eidx, sem = bset

        @pl.when(jnp.any(iloc_r[...] != DUMMY))
        def _go():
            pltpu.make_async_copy(phiv.at[pl.ds(0, K)], pv, sem).wait()
            pltpu.make_async_copy(wmat.at[pl.ds(0, K)], wm, sem).wait()

            @plsc.parallel_loop(0, K, 1, unroll=1)
            def _edge(q):
                qv = jnp.full((16,), q, jnp.int32)
                rb = plsc.load_gather(iloc_r, [qv])
                uq = [plsc.load_gather(wm, [qv, jnp.full((16,), F3 + d, jnp.int32)])
                      for d in range(3)]
                vals = []
                for t in range(8):
                    vals.append((F3 + 16 * t + iota16,
                                 pv[q, pl.ds(F + 16 * t, 16)]
                                 * wm[q, pl.ds(F + 16 * t, 16)]))
                for t in range(8):
                    fidx = 16 * t + iota16
                    s1t = pv[q, pl.ds(16 * t, 16)] * wm[q, pl.ds(16 * t, 16)]
                    t3 = (pv[q, pl.ds(2 * F + 16 * t, 16)]
                          * wm[q, pl.ds(2 * F + 16 * t, 16)])
                    for d in range(3):
                        vals.append((3 * fidx + d,
                                     s1t * pv[q, pl.ds(F3 + d * F + 16 * t, 16)]
                                     + t3 * uq[d]))
                    if len(vals) >= 14:
                        for cidx, val in vals:
                            plsc.addupdate_scatter(slab, [rb, cidx], val)
                        vals = []
                for cidx, val in vals:
                    plsc.addupdate_scatter(slab, [rb, cidx], val)

    def _drain(bset):
        pv, wm, iloc_r, _, _, sem = bset

        @pl.when(jnp.any(iloc_r[...] != DUMMY))
        def _():
            pltpu.make_async_copy(phiv.at[pl.ds(0, K)], pv, sem).wait()
            pltpu.make_async_copy(wmat.at[pl.ds(0, K)], wm, sem).wait()

    def _pass(p, _0):
        lo = (wid * P + p) * RT

        @plsc.parallel_loop(0, SLAB_ROWS * (CW // 16), 1, unroll=4)
        def _zs(t):
            slab[t // (CW // 16), pl.ds((t % (CW // 16)) * 16, 16)] = zero16

        def _stage(ch, cbuf, jcbuf, semc):
            cb = jnp.minimum(ch, E // CE - 1) * CE
            pltpu.async_copy(it_h.at[pl.ds(cb, CE)], cbuf, semc)
            pltpu.async_copy(jt_h.at[pl.ds(cb, CE)], jcbuf, semc)

        def _stage_wait(cbuf, jcbuf, semc):
            pltpu.make_async_copy(it_h.at[pl.ds(0, CE)], cbuf, semc).wait()
            pltpu.make_async_copy(jt_h.at[pl.ds(0, CE)], jcbuf, semc).wait()

        def _one_chunk(ch, cbuf, jcbuf):
            cb = ch * CE

            @plsc.parallel_loop(0, CE // 16, 1, unroll=2,
                                carry=jnp.zeros((16,), jnp.int32))
            def tot(t, tot_c):
                ii = cbuf[pl.ds(t * 16, 16)]
                m = (ii >= lo) & (ii < lo + RT)
                pref = plsc.cumsum(m.astype(jnp.int32))
                plsc.store_scatter(sel, [tot_c + pref - 1], t * 16 + iota16, mask=m)
                return tot_c + plsc.all_reduce_population_count(m)
            neg1 = jnp.full((16,), -1, jnp.int32)
            for w in range(4):
                plsc.store_scatter(sel, [tot + 16 * w + iota16], neg1)

            _drain(bufs[0])
            _prep(0, cb, lo, bufs[0], cbuf, jcbuf)

            def _batch_cond(carry):
                return carry[1]

            def _batch(carry):
                base, _ = carry
                _prep(base + K, cb, lo, bufs[1], cbuf, jcbuf)
                _compute(bufs[0])
                _prep(base + 2 * K, cb, lo, bufs[0], cbuf, jcbuf)
                _compute(bufs[1])
                cont = (jnp.any(sel[pl.ds(base + 2 * K, 16)] >= 0)
                        | jnp.any(sel[pl.ds(base + 3 * K, 16)] >= 0))
                return base + 2 * K, cont

            lax.while_loop(_batch_cond, _batch, (jnp.int32(0), jnp.bool_(True)))

        def _chunk2(k, _):
            ch = 2 * k
            _stage_wait(cbufa, jcbufa, semca)
            _stage(ch + 1, cbufb, jcbufb, semcb)
            _one_chunk(ch, cbufa, jcbufa)
            _stage_wait(cbufb, jcbufb, semcb)
            _stage(ch + 2, cbufa, jcbufa, semca)
            _one_chunk(ch + 1, cbufb, jcbufb)
            return 0

        jidxa[...] = jnp.zeros((16,), jnp.int32)
        eidxa[...] = jnp.zeros((16,), jnp.int32)
        iloca[...] = jnp.zeros((16,), jnp.int32)
        pltpu.async_copy(phiv.at[jidxa], pva, sema)
        pltpu.async_copy(wmat.at[eidxa], wma, sema)
        _stage(0, cbufa, jcbufa, semca)
        lax.fori_loop(0, E // CE // 2, _chunk2, 0)
        _stage_wait(cbufa, jcbufa, semca)

        cps = []
        for t in range(RT // 16):
            cps.append(pltpu.async_copy(slab.at[pl.ds(t * 16, 16)],
                                        out.at[pl.ds(lo + t * 16, 16)], semf))
        _drain(bufs[0])
        for cp in cps:
            cp.wait()
        return 0

    lax.fori_loop(0, P, _pass, 0)


def _sc_edges(phiv, wmat, it_, jt_):
    f32, i32 = jnp.float32, jnp.int32
    mesh = plsc.VectorSubcoreMesh(core_axis_name="c", subcore_axis_name="s")
    fn = pl.kernel(
        _sc_edges_body,
        out_type=jax.ShapeDtypeStruct((NOUT, CW), f32),
        mesh=mesh,
        compiler_params=pltpu.CompilerParams(needs_layout_passes=False),
        scratch_types=[
            pltpu.VMEM((SLAB_ROWS, CW), f32),
            pltpu.VMEM((CE,), i32),
            pltpu.VMEM((CE,), i32),
            pltpu.VMEM((CE,), i32),
            pltpu.VMEM((CE,), i32),
            pltpu.VMEM((CE + 80,), i32),
            pltpu.VMEM((K, WMW), f32),
            pltpu.VMEM((K, WMW), f32),
            pltpu.VMEM((K, WM2), f32),
            pltpu.VMEM((K, WM2), f32),
            pltpu.VMEM((K,), i32),
            pltpu.VMEM((K,), i32),
            pltpu.VMEM((K,), i32),
            pltpu.VMEM((K,), i32),
            pltpu.VMEM((K,), i32),
            pltpu.VMEM((K,), i32),
            pltpu.SemaphoreType.DMA,
            pltpu.SemaphoreType.DMA,
            pltpu.SemaphoreType.DMA,
            pltpu.SemaphoreType.DMA,
            pltpu.SemaphoreType.DMA,
        ],
    )
    return fn(phiv, wmat, it_, jt_)


def kernel(v, s, edges_indices, edges_diff, edges_norm, W1, b1, W2, b2, Wr, br):
    vt = v.transpose(2, 0, 1)
    phiv = _compute_phiv(s, W1, b1, W2, b2, vt[0], vt[1], vt[2])
    wmat = _compute_wmat(edges_norm, edges_diff, Wr, br)
    it_ = edges_indices[:, 0]
    jt_ = edges_indices[:, 1]
    out = _sc_edges(phiv, wmat, it_, jt_)
    dv = out[:N, :F3].reshape(N, F, 3)
    ds = out[:N, F3:]
    return (dv, ds)


# speedup vs baseline: 1.7437x; 1.0019x over previous
import functools

import jax
import jax.numpy as jnp
import numpy as np
from jax import lax
from jax.experimental import pallas as pl
from jax.experimental.pallas import tpu as pltpu
from jax.experimental.pallas import tpu_sc as plsc

F = 128
F3 = 384
NRBF = 20
CUT = 5.0
N = 10000
E = 160000

NC, NS = 2, 16
P = 3
NW = NC * NS
RT = 112
NOUT = P * NW * RT
SLAB_ROWS = 128
DUMMY = RT
CE = 4000
K = 16
CW = F3 + F
WMW = F3 + 3 * F
WM2 = F3 + F


def _phiv_body(s_ref, w1_ref, b1_ref, w2_ref, b2_ref, vx_ref, vy_ref, vz_ref,
               out_ref):
    x = s_ref[...]
    h = jnp.dot(x, w1_ref[...], preferred_element_type=jnp.float32) + b1_ref[...]
    h = h * jax.nn.sigmoid(h)
    out_ref[:, :F3] = (
        jnp.dot(h, w2_ref[...], preferred_element_type=jnp.float32) + b2_ref[...]
    )
    out_ref[:, F3:F3 + F] = vx_ref[...]
    out_ref[:, F3 + F:F3 + 2 * F] = vy_ref[...]
    out_ref[:, F3 + 2 * F:] = vz_ref[...]


def _compute_phiv(s, W1, b1, W2, b2, vtx, vty, vtz):
    blk = 2000
    return pl.pallas_call(
        _phiv_body,
        grid=(N // blk,),
        in_specs=[
            pl.BlockSpec((blk, F), lambda i: (i, 0)),
            pl.BlockSpec((F, F), lambda i: (0, 0)),
            pl.BlockSpec((1, F), lambda i: (0, 0)),
            pl.BlockSpec((F, F3), lambda i: (0, 0)),
            pl.BlockSpec((1, F3), lambda i: (0, 0)),
            pl.BlockSpec((blk, F), lambda i: (i, 0)),
            pl.BlockSpec((blk, F), lambda i: (i, 0)),
            pl.BlockSpec((blk, F), lambda i: (i, 0)),
        ],
        out_specs=pl.BlockSpec((blk, WMW), lambda i: (i, 0)),
        out_shape=jax.ShapeDtypeStruct((N, WMW), jnp.float32),
    )(s, W1, b1.reshape(1, F), W2, b2.reshape(1, F3), vtx, vty, vtz)


def _wmat_body(en_ref, diff_ref, wr_ref, br_ref, out_ref):
    en = en_ref[0, 0, :]
    nk = (lax.broadcasted_iota(jnp.int32, (1, NRBF), 1) + 1).astype(
        jnp.float32) * (np.pi / CUT)
    inv = 1.0 / en
    rbf = jnp.sin(en[:, None] * nk) * inv[:, None]
    wm = jnp.dot(rbf, wr_ref[...], preferred_element_type=jnp.float32) + br_ref[...]
    fcut = 0.5 * (jnp.cos(en * (np.pi / CUT)) + 1.0)
    wm = wm * fcut[:, None]
    out_ref[:, :F3] = wm
    blk = en.shape[0]
    u3 = diff_ref[...] * inv[:, None]
    out_ref[:, F3:] = jnp.concatenate(
        [u3, jnp.zeros((blk, F - 3), jnp.float32)], axis=1)


def _compute_wmat(edges_norm, edges_diff, Wr, br):
    blk = 2000
    return pl.pallas_call(
        _wmat_body,
        grid=(E // blk,),
        in_specs=[
            pl.BlockSpec((1, 1, blk), lambda i: (i, 0, 0)),
            pl.BlockSpec((blk, 3), lambda i: (i, 0)),
            pl.BlockSpec((NRBF, F3), lambda i: (0, 0)),
            pl.BlockSpec((1, F3), lambda i: (0, 0)),
        ],
        out_specs=pl.BlockSpec((blk, WM2), lambda i: (i, 0)),
        out_shape=jax.ShapeDtypeStruct((E, WM2), jnp.float32),
    )(edges_norm.reshape(E // blk, 1, blk), edges_diff, Wr, br.reshape(1, F3))


def _sc_edges_body(phiv, wmat, it_h, jt_h, out,
                   slab, cbufa, jcbufa, cbufb, jcbufb, sel,
                   pva, pvb, wma, wmb, iloca, ilocb,
                   jidxa, jidxb, eidxa, eidxb, sema, semb, semf,
                   semca, semcb):
    c = lax.axis_index("c")
    s = lax.axis_index("s")
    wid = s * NC + c
    iota16 = lax.broadcasted_iota(jnp.int32, (16,), 0)
    zero16 = jnp.zeros((16,), jnp.float32)

    bufs = ((pva, wma, iloca, jidxa, eidxa, sema),
            (pvb, wmb, ilocb, jidxb, eidxb, semb))

    def _prep(b, cb, lo, bset, cbuf, jcbuf):
        pv, wm, iloc_r, jidx, eidx, sem = bset
        ecand = sel[pl.ds(b, 16)]
        valid = ecand >= 0
        e16 = jnp.where(valid, ecand, 0)
        j16 = plsc.load_gather(jcbuf, [e16])
        i16 = plsc.load_gather(cbuf, [e16])
        jidx[...] = j16
        eidx[...] = e16 + cb
        iloc_r[...] = jnp.where(valid, i16 - lo, DUMMY)

        @pl.when(jnp.any(valid))
        def _():
            pltpu.async_copy(phiv.at[jidx], pv, sem)
            pltpu.async_copy(wmat.at[eidx], wm, sem)

    def _compute(bset):
        pv, wm, iloc_r, jidx, eidx, sem = bset

        @pl.when(jnp.any(iloc_r[...] != DUMMY))
        def _go():
            pltpu.make_async_copy(phiv.at[pl.ds(0, K)], pv, sem).wait()
            pltpu.make_async_copy(wmat.at[pl.ds(0, K)], wm, sem).wait()

            @plsc.parallel_loop(0, K, 1, unroll=1)
            def _edge(q):
                qv = jnp.full((16,), q, jnp.int32)
                rb = plsc.load_gather(iloc_r, [qv])
                uq = [plsc.load_gather(wm, [qv, jnp.full((16,), F3 + d, jnp.int32)])
                      for d in range(3)]
                vals = []
                for t in range(8):
                    vals.append((F3 + 16 * t + iota16,
                                 pv[q, pl.ds(F + 16 * t, 16)]
                                 * wm[q, pl.ds(F + 16 * t, 16)]))
                for t in range(8):
                    fidx = 16 * t + iota16
                    s1t = pv[q, pl.ds(16 * t, 16)] * wm[q, pl.ds(16 * t, 16)]
                    t3 = (pv[q, pl.ds(2 * F + 16 * t, 16)]
                          * wm[q, pl.ds(2 * F + 16 * t, 16)])
                    for d in range(3):
                        vals.append((3 * fidx + d,
                                     s1t * pv[q, pl.ds(F3 + d * F + 16 * t, 16)]
                                     + t3 * uq[d]))
                    if len(vals) >= 20:
                        for cidx, val in vals:
                            plsc.addupdate_scatter(slab, [rb, cidx], val)
                        vals = []
                for cidx, val in vals:
                    plsc.addupdate_scatter(slab, [rb, cidx], val)

    def _drain(bset):
        pv, wm, iloc_r, _, _, sem = bset

        @pl.when(jnp.any(iloc_r[...] != DUMMY))
        def _():
            pltpu.make_async_copy(phiv.at[pl.ds(0, K)], pv, sem).wait()
            pltpu.make_async_copy(wmat.at[pl.ds(0, K)], wm, sem).wait()

    def _pass(p, _0):
        lo = (wid * P + p) * RT

        @plsc.parallel_loop(0, SLAB_ROWS * (CW // 16), 1, unroll=4)
        def _zs(t):
            slab[t // (CW // 16), pl.ds((t % (CW // 16)) * 16, 16)] = zero16

        def _stage(ch, cbuf, jcbuf, semc):
            cb = jnp.minimum(ch, E // CE - 1) * CE
            pltpu.async_copy(it_h.at[pl.ds(cb, CE)], cbuf, semc)
            pltpu.async_copy(jt_h.at[pl.ds(cb, CE)], jcbuf, semc)

        def _stage_wait(cbuf, jcbuf, semc):
            pltpu.make_async_copy(it_h.at[pl.ds(0, CE)], cbuf, semc).wait()
            pltpu.make_async_copy(jt_h.at[pl.ds(0, CE)], jcbuf, semc).wait()

        def _one_chunk(ch, cbuf, jcbuf):
            cb = ch * CE

            @plsc.parallel_loop(0, CE // 16, 1, unroll=4,
                                carry=jnp.zeros((16,), jnp.int32))
            def tot(t, tot_c):
                ii = cbuf[pl.ds(t * 16, 16)]
                m = (ii >= lo) & (ii < lo + RT)
                pref = plsc.cumsum(m.astype(jnp.int32))
                plsc.store_scatter(sel, [tot_c + pref - 1], t * 16 + iota16, mask=m)
                return tot_c + plsc.all_reduce_population_count(m)
            neg1 = jnp.full((16,), -1, jnp.int32)
            for w in range(4):
                plsc.store_scatter(sel, [tot + 16 * w + iota16], neg1)

            _drain(bufs[0])
            _prep(0, cb, lo, bufs[0], cbuf, jcbuf)

            def _batch_cond(carry):
                return carry[1]

            def _batch(carry):
                base, _ = carry
                _prep(base + K, cb, lo, bufs[1], cbuf, jcbuf)
                _compute(bufs[0])
                _prep(base + 2 * K, cb, lo, bufs[0], cbuf, jcbuf)
                _compute(bufs[1])
                cont = (jnp.any(sel[pl.ds(base + 2 * K, 16)] >= 0)
                        | jnp.any(sel[pl.ds(base + 3 * K, 16)] >= 0))
                return base + 2 * K, cont

            lax.while_loop(_batch_cond, _batch, (jnp.int32(0), jnp.bool_(True)))

        def _chunk2(k, _):
            ch = 2 * k
            _stage_wait(cbufa, jcbufa, semca)
            _stage(ch + 1, cbufb, jcbufb, semcb)
            _one_chunk(ch, cbufa, jcbufa)
            _stage_wait(cbufb, jcbufb, semcb)
            _stage(ch + 2, cbufa, jcbufa, semca)
            _one_chunk(ch + 1, cbufb, jcbufb)
            return 0

        jidxa[...] = jnp.zeros((16,), jnp.int32)
        eidxa[...] = jnp.zeros((16,), jnp.int32)
        iloca[...] = jnp.zeros((16,), jnp.int32)
        pltpu.async_copy(phiv.at[jidxa], pva, sema)
        pltpu.async_copy(wmat.at[eidxa], wma, sema)
        _stage(0, cbufa, jcbufa, semca)
        lax.fori_loop(0, E // CE // 2, _chunk2, 0)
        _stage_wait(cbufa, jcbufa, semca)

        cps = []
        for t in range(RT // 16):
            cps.append(pltpu.async_copy(slab.at[pl.ds(t * 16, 16)],
                                        out.at[pl.ds(lo + t * 16, 16)], semf))
        _drain(bufs[0])
        for cp in cps:
            cp.wait()
        return 0

    lax.fori_loop(0, P, _pass, 0)


def _sc_edges(phiv, wmat, it_, jt_):
    f32, i32 = jnp.float32, jnp.int32
    mesh = plsc.VectorSubcoreMesh(core_axis_name="c", subcore_axis_name="s")
    fn = pl.kernel(
        _sc_edges_body,
        out_type=jax.ShapeDtypeStruct((NOUT, CW), f32),
        mesh=mesh,
        compiler_params=pltpu.CompilerParams(needs_layout_passes=False),
        scratch_types=[
            pltpu.VMEM((SLAB_ROWS, CW), f32),
            pltpu.VMEM((CE,), i32),
            pltpu.VMEM((CE,), i32),
            pltpu.VMEM((CE,), i32),
            pltpu.VMEM((CE,), i32),
            pltpu.VMEM((CE + 80,), i32),
            pltpu.VMEM((K, WMW), f32),
            pltpu.VMEM((K, WMW), f32),
            pltpu.VMEM((K, WM2), f32),
            pltpu.VMEM((K, WM2), f32),
            pltpu.VMEM((K,), i32),
            pltpu.VMEM((K,), i32),
            pltpu.VMEM((K,), i32),
            pltpu.VMEM((K,), i32),
            pltpu.VMEM((K,), i32),
            pltpu.VMEM((K,), i32),
            pltpu.SemaphoreType.DMA,
            pltpu.SemaphoreType.DMA,
            pltpu.SemaphoreType.DMA,
            pltpu.SemaphoreType.DMA,
            pltpu.SemaphoreType.DMA,
        ],
    )
    return fn(phiv, wmat, it_, jt_)


def kernel(v, s, edges_indices, edges_diff, edges_norm, W1, b1, W2, b2, Wr, br):
    vt = v.transpose(2, 0, 1)
    phiv = _compute_phiv(s, W1, b1, W2, b2, vt[0], vt[1], vt[2])
    wmat = _compute_wmat(edges_norm, edges_diff, Wr, br)
    it_ = edges_indices[:, 0]
    jt_ = edges_indices[:, 1]
    out = _sc_edges(phiv, wmat, it_, jt_)
    dv = out[:N, :F3].reshape(N, F, 3)
    ds = out[:N, F3:]
    return (dv, ds)
